# baseline (device time: 35337 ns/iter reference)
import jax
import jax.numpy as jnp
from jax import lax
from jax.experimental import pallas as pl
from jax.experimental.pallas import tpu as pltpu

N_CHUNKS = 8


def kernel(partial, resid, gamma):
    _, m, d = partial.shape
    partial2d = partial.reshape(m, d)
    gamma2d = gamma.reshape(1, d)
    rows = m // N_CHUNKS

    def body(partial_ref, resid_ref, gamma_ref, out_ref,
             send_buf, recv_buf, send_sems, recv_sems):
        my_x = lax.axis_index("x")
        my_y = lax.axis_index("y")
        my_z = lax.axis_index("z")
        partner = (my_x, my_y, 1 - my_z)

        barrier_sem = pltpu.get_barrier_semaphore()
        pl.semaphore_signal(
            barrier_sem, inc=1,
            device_id=partner, device_id_type=pl.DeviceIdType.MESH,
        )
        pl.semaphore_wait(barrier_sem, 1)

        rdmas = []
        for c in range(N_CHUNKS):
            sl = pl.ds(c * rows, rows)
            send_buf[sl, :] = partial_ref[sl, :].astype(jnp.bfloat16)
            rdma = pltpu.make_async_remote_copy(
                src_ref=send_buf.at[sl, :],
                dst_ref=recv_buf.at[sl, :],
                send_sem=send_sems.at[c],
                recv_sem=recv_sems.at[c],
                device_id=partner,
                device_id_type=pl.DeviceIdType.MESH,
            )
            rdma.start()
            rdmas.append(rdma)

        for c in range(N_CHUNKS):
            sl = pl.ds(c * rows, rows)
            rdmas[c].wait_recv()
            y = (partial_ref[sl, :] + recv_buf[sl, :].astype(jnp.float32)
                 + resid_ref[sl, :])
            rms = jnp.sqrt(jnp.mean(y * y, axis=-1, keepdims=True) + 1e-6)
            out_ref[sl, :] = y / rms * gamma_ref[...]

        for c in range(N_CHUNKS):
            rdmas[c].wait_send()

    return pl.pallas_call(
        body,
        out_shape=jax.ShapeDtypeStruct((m, d), jnp.float32),
        in_specs=[
            pl.BlockSpec(memory_space=pltpu.VMEM),
            pl.BlockSpec(memory_space=pltpu.VMEM),
            pl.BlockSpec(memory_space=pltpu.VMEM),
        ],
        out_specs=pl.BlockSpec(memory_space=pltpu.VMEM),
        scratch_shapes=[
            pltpu.VMEM((m, d), jnp.bfloat16),
            pltpu.VMEM((m, d), jnp.bfloat16),
            pltpu.SemaphoreType.DMA((N_CHUNKS,)),
            pltpu.SemaphoreType.DMA((N_CHUNKS,)),
        ],
        compiler_params=pltpu.CompilerParams(collective_id=0),
    )(partial2d, resid, gamma2d)


# device time: 23929 ns/iter; 1.4767x vs baseline; 1.4767x over previous
import jax
import jax.numpy as jnp
from jax import lax
from jax.experimental import pallas as pl
from jax.experimental.pallas import tpu as pltpu

CH = 64


def kernel(partial, resid, gamma):
    _, m, d = partial.shape
    partial2d = partial.reshape(m, d)
    gamma2d = gamma.reshape(1, d)
    qrows = m // 4
    C = qrows // CH

    def body(partial_ref, resid_ref, gamma_ref, out_ref,
             other_buf, sendz, zr, xr, yr, zs, xs, ys):
        my_x = lax.axis_index("x")
        my_y = lax.axis_index("y")
        my_z = lax.axis_index("z")
        q = 2 * my_x + my_y
        qd = 3 - q
        qx = 2 * (1 - my_x) + my_y
        qy = 2 * my_x + (1 - my_y)
        zdev = (my_x, my_y, 1 - my_z)
        xdev = (1 - my_x, my_y, my_z)
        ydev = (my_x, 1 - my_y, my_z)

        barrier_sem = pltpu.get_barrier_semaphore()
        for dev in (zdev, xdev, ydev):
            pl.semaphore_signal(
                barrier_sem, inc=1,
                device_id=dev, device_id_type=pl.DeviceIdType.MESH,
            )
        pl.semaphore_wait(barrier_sem, 3)

        def compute(row0):
            sl = pl.ds(row0, CH)
            y = (partial_ref[sl, :] + other_buf[sl, :].astype(jnp.float32)
                 + resid_ref[sl, :])
            rms = jnp.sqrt(jnp.mean(y * y, axis=-1, keepdims=True) + 1e-6)
            out_ref[sl, :] = y / rms * gamma_ref[...]

        z_rdmas = []
        for c in range(2 * C):
            qq = q if c < C else qd
            row0 = qq * qrows + (c % C) * CH
            loc0 = c * CH
            sendz[pl.ds(loc0, CH), :] = (
                partial_ref[pl.ds(row0, CH), :].astype(jnp.bfloat16))
            rdma = pltpu.make_async_remote_copy(
                src_ref=sendz.at[pl.ds(loc0, CH), :],
                dst_ref=other_buf.at[pl.ds(row0, CH), :],
                send_sem=zs.at[c], recv_sem=zr.at[c],
                device_id=zdev, device_id_type=pl.DeviceIdType.MESH,
            )
            rdma.start()
            z_rdmas.append(rdma)

        xy_rdmas = []
        for c in range(C):
            z_rdmas[c].wait_recv()
            row0 = q * qrows + c * CH
            sl = pl.ds(row0, CH)
            for dev, ss, rr in ((xdev, xs, xr), (ydev, ys, yr)):
                r = pltpu.make_async_remote_copy(
                    src_ref=other_buf.at[sl, :],
                    dst_ref=other_buf.at[sl, :],
                    send_sem=ss.at[c], recv_sem=rr.at[c],
                    device_id=dev, device_id_type=pl.DeviceIdType.MESH,
                )
                r.start()
                xy_rdmas.append(r)
            compute(row0)

        for c in range(C):
            for qq, rr in ((qx, xr), (qy, yr)):
                row0 = qq * qrows + c * CH
                sl = pl.ds(row0, CH)
                pltpu.make_async_remote_copy(
                    src_ref=other_buf.at[sl, :],
                    dst_ref=other_buf.at[sl, :],
                    send_sem=zs.at[0], recv_sem=rr.at[c],
                    device_id=zdev, device_id_type=pl.DeviceIdType.MESH,
                ).wait_recv()
                compute(row0)

        for c in range(C, 2 * C):
            z_rdmas[c].wait_recv()
            compute(qd * qrows + (c % C) * CH)

        for r in z_rdmas + xy_rdmas:
            r.wait_send()

    return pl.pallas_call(
        body,
        out_shape=jax.ShapeDtypeStruct((m, d), jnp.float32),
        in_specs=[
            pl.BlockSpec(memory_space=pltpu.VMEM),
            pl.BlockSpec(memory_space=pltpu.VMEM),
            pl.BlockSpec(memory_space=pltpu.VMEM),
        ],
        out_specs=pl.BlockSpec(memory_space=pltpu.VMEM),
        scratch_shapes=[
            pltpu.VMEM((m, d), jnp.bfloat16),
            pltpu.VMEM((m // 2, d), jnp.bfloat16),
            pltpu.SemaphoreType.DMA((2 * C,)),
            pltpu.SemaphoreType.DMA((C,)),
            pltpu.SemaphoreType.DMA((C,)),
            pltpu.SemaphoreType.DMA((2 * C,)),
            pltpu.SemaphoreType.DMA((C,)),
            pltpu.SemaphoreType.DMA((C,)),
        ],
        compiler_params=pltpu.CompilerParams(collective_id=0),
    )(partial2d, resid, gamma2d)
